# Initial kernel scaffold; baseline (speedup 1.0000x reference)
import jax
import jax.numpy as jnp
from jax.experimental import pallas as pl


def kernel(x, edge_index, edge_attr, W_node, b_node, W_eenc, b_eenc, W0, att_src0, att_dst0, We0, att_edge0, bias0, bn_w0, bn_b0, W1, att_src1, att_dst1, We1, att_edge1, bias1, bn_w1, bn_b1, W_out, b_out):
    raise NotImplementedError("write your pallas kernel here")


# speedup vs baseline: 1.6525x; 1.6525x over previous
import functools

import jax
import jax.numpy as jnp
from jax.experimental import pallas as pl

N = 10000
E = 320000
EPS_BN = 1e-5


def _leaky(v):
    return jnp.where(v >= 0, v, 0.2 * v)


def _proj_body(p_ref, w_ref, b_ref, o_ref):
    o_ref[...] = p_ref[...] @ w_ref[...] + b_ref[...]


def _proj(pooled, W_out, b_out):
    out = pl.pallas_call(
        _proj_body,
        out_shape=jax.ShapeDtypeStruct((1, W_out.shape[1]), jnp.float32),
    )(pooled.reshape(1, -1), W_out, b_out.reshape(1, -1))
    return out.reshape(-1)


def _layer(h, src, dst, aer, W, att_src, att_dst, bias, gw, gb):
    xs = h @ W
    asrc = xs @ att_src
    adst = xs @ att_dst
    deg = jax.ops.segment_sum(jnp.ones((E,), jnp.float32), dst, num_segments=N)
    s = jax.ops.segment_sum(aer, dst, num_segments=N)
    p = jnp.exp(_leaky(asrc[src] + adst[dst] + aer))
    den = jax.ops.segment_sum(p, dst, num_segments=N)
    num = jax.ops.segment_sum(xs[src] * p[:, None], dst, num_segments=N)
    p_loop = jnp.exp(_leaky(asrc + adst + s / jnp.maximum(deg, 1.0)))
    out = (num + p_loop[:, None] * xs) / (den + p_loop + 1e-16)[:, None] + bias
    inv = 1.0 / jnp.sqrt(1.0 + EPS_BN)
    return jax.nn.relu(out * inv * gw + gb)


def kernel(x, edge_index, edge_attr, W_node, b_node, W_eenc, b_eenc,
           W0, att_src0, att_dst0, We0, att_edge0, bias0, bn_w0, bn_b0,
           W1, att_src1, att_dst1, We1, att_edge1, bias1, bn_w1, bn_b1,
           W_out, b_out):
    src = edge_index[0]
    dst = edge_index[1]
    we0 = We0 @ att_edge0
    we1 = We1 @ att_edge1
    aer0 = edge_attr @ (W_eenc @ we0) + b_eenc @ we0
    aer1 = edge_attr @ (W_eenc @ we1) + b_eenc @ we1
    h = x @ W_node + b_node
    h = _layer(h, src, dst, aer0, W0, att_src0, att_dst0, bias0, bn_w0, bn_b0)
    h = _layer(h, src, dst, aer1, W1, att_src1, att_dst1, bias1, bn_w1, bn_b1)
    pooled = jnp.mean(h, axis=0)
    return _proj(pooled, W_out, b_out)


# speedup vs baseline: 16.7878x; 10.1589x over previous
import functools

import jax
import jax.numpy as jnp
from jax import lax
from jax.experimental import pallas as pl
from jax.experimental.pallas import tpu as pltpu
from jax.experimental.pallas import tpu_sc as plsc

N = 10000
E = 320000
NP = 10240
NW = 32
EW = 10112
B = 128
ROWS_PER_SUB = NP // 16
INV_STD = 1.0 / (1.0 + 1e-5) ** 0.5


def _mm_body(x_ref, w_ref, b_ref, o_ref):
    o_ref[...] = (
        jnp.dot(x_ref[...], w_ref[...], preferred_element_type=jnp.float32)
        + b_ref[...]
    )


def _mm(x, w, b, blk):
    m, k = x.shape
    n = w.shape[1]
    return pl.pallas_call(
        _mm_body,
        grid=(m // blk,),
        in_specs=[
            pl.BlockSpec((blk, k), lambda i: (i, 0)),
            pl.BlockSpec((k, n), lambda i: (0, 0)),
            pl.BlockSpec((1, n), lambda i: (0, 0)),
        ],
        out_specs=pl.BlockSpec((blk, n), lambda i: (i, 0)),
        out_shape=jax.ShapeDtypeStruct((m, n), jnp.float32),
    )(x, w, b.reshape(1, n))


def _leaky_exp(a):
    return jnp.exp(jnp.where(a >= 0, a, 0.2 * a))


def _merge_body(numA, numB, denA, denB, degA, degB, s0A, s0B, s1A, s1B,
                asrc, adst, xs0, wc, bc, bias0, gw, gb, o_ref):
    num = numA[...] + numB[...]
    den = denA[...] + denB[...]
    deg = jnp.maximum(degA[...] + degB[...], 1.0)
    s0 = s0A[...] + s0B[...]
    s1 = s1A[...] + s1B[...]
    pl0 = _leaky_exp(asrc[...] + adst[...] + s0 / deg)
    out0 = (num + pl0 * xs0[...]) / (den + pl0 + 1e-16) + bias0[...]
    h1 = jnp.maximum(out0 * INV_STD * gw[...] + gb[...], 0.0)
    m = jnp.dot(h1, wc[...], preferred_element_type=jnp.float32) + bc[...]
    a1 = m[:, 128:129] + m[:, 129:130] + s1 / deg
    pl1 = _leaky_exp(a1)
    o_ref[...] = jnp.concatenate([m[:, :130], pl1, m[:, 131:]], axis=1)


def _final_body(numA, numB, denA, denB, pl1, xs1, bias1, gw, gb, wo, bo,
                o_ref, acc_ref):
    i = pl.program_id(0)
    num = numA[...] + numB[...]
    den = denA[...] + denB[...]
    p = pl1[...]
    out1 = (num + p * xs1[...]) / (den + p + 1e-16) + bias1[...]
    h2 = jnp.maximum(out1 * INV_STD * gw[...] + gb[...], 0.0)
    part = jnp.sum(h2, axis=0, keepdims=True)

    @pl.when(i == 0)
    def _():
        acc_ref[...] = jnp.zeros_like(acc_ref)

    acc_ref[0:1, :] += part

    @pl.when(i == pl.num_programs(0) - 1)
    def _():
        pooled = acc_ref[0:1, :] / float(N)
        o_ref[...] = (
            jnp.dot(pooled, wo[...], preferred_element_type=jnp.float32)
            + bo[...]
        )


def _sc_body(with_extras, *refs):
    if with_extras:
        (xs, asrc_h, adst_h, src_h, dst_h, aer_h, aer2_h,
         num_o, den_o, deg_o, s0_o, s1_o,
         asrc_v, adst_v, srcb, dstb, aerb, aer2b, pb, onesb, rows,
         acc_rows, acc_den, acc_deg, acc_s0, acc_s1) = refs
        scalar_accs = (acc_den, acc_deg, acc_s0, acc_s1)
        scalar_outs = (den_o, deg_o, s0_o, s1_o)
    else:
        (xs, asrc_h, adst_h, src_h, dst_h, aer_h,
         num_o, den_o,
         asrc_v, adst_v, srcb, dstb, aerb, pb, onesb, rows,
         acc_rows, acc_den) = refs
        aer2_h = aer2b = None
        scalar_accs = (acc_den,)
        scalar_outs = (den_o,)

    c = lax.axis_index("c")
    s = lax.axis_index("s")
    w = c * 16 + s
    zeros16 = jnp.zeros((16,), jnp.float32)
    ones16 = jnp.ones((16,), jnp.float32)

    def zrow(r, _):
        for k in range(8):
            rows[r, pl.ds(k * 16, 16)] = zeros16
        return 0
    lax.fori_loop(0, B, zrow, 0)
    for j in range(B // 16):
        onesb[pl.ds(j * 16, 16)] = ones16

    base = s * ROWS_PER_SUB
    for k in range(ROWS_PER_SUB // B):
        sl = pl.ds(base + k * B, B)
        pltpu.sync_copy(rows, acc_rows.at[sl])
        for acc in scalar_accs:
            pltpu.sync_copy(rows.at[0], acc.at[sl])

    pltpu.sync_copy(asrc_h, asrc_v)
    pltpu.sync_copy(adst_h, adst_v)
    plsc.subcore_barrier()

    ebase = w * EW
    nb = jnp.where(w == NW - 1, (E - (NW - 1) * EW) // B, EW // B)

    def batch(b, _):
        off = ebase + b * B
        pltpu.sync_copy(src_h.at[pl.ds(off, B)], srcb)
        pltpu.sync_copy(dst_h.at[pl.ds(off, B)], dstb)
        pltpu.sync_copy(aer_h.at[pl.ds(off, B)], aerb)
        if with_extras:
            pltpu.sync_copy(aer2_h.at[pl.ds(off, B)], aer2b)
        pltpu.sync_copy(xs.at[srcb], rows)

        def pj(j, _):
            s16 = srcb[pl.ds(j * 16, 16)]
            d16 = dstb[pl.ds(j * 16, 16)]
            a = (plsc.load_gather(asrc_v, [s16])
                 + plsc.load_gather(adst_v, [d16])
                 + aerb[pl.ds(j * 16, 16)])
            a = jnp.where(a >= 0, a, 0.2 * a)
            pb[pl.ds(j * 16, 16)] = jnp.exp(a)
            return 0
        lax.fori_loop(0, B // 16, pj, 0)

        def scale(e, _):
            pe = plsc.load_gather(pb, [jnp.full((16,), e, jnp.int32)])
            for k in range(8):
                sl = pl.ds(k * 16, 16)
                rows[e, sl] = rows[e, sl] * pe
            return 0
        lax.fori_loop(0, B, scale, 0)

        pltpu.sync_copy(rows, acc_rows.at[dstb], add=True)
        pltpu.sync_copy(pb, acc_den.at[dstb], add=True)
        if with_extras:
            pltpu.sync_copy(onesb, acc_deg.at[dstb], add=True)
            pltpu.sync_copy(aerb, acc_s0.at[dstb], add=True)
            pltpu.sync_copy(aer2b, acc_s1.at[dstb], add=True)
        return 0
    lax.fori_loop(0, nb, batch, 0)
    plsc.subcore_barrier()

    for k in range(ROWS_PER_SUB // B):
        sl = pl.ds(base + k * B, B)
        pltpu.sync_copy(acc_rows.at[sl], num_o.at[c, sl])
        for acc, out in zip(scalar_accs, scalar_outs):
            pltpu.sync_copy(acc.at[sl], out.at[c, sl])


def _sc_pass(with_extras):
    mesh = plsc.VectorSubcoreMesh(core_axis_name="c", subcore_axis_name="s")
    out_type = [jax.ShapeDtypeStruct((2, NP, 128), jnp.float32),
                jax.ShapeDtypeStruct((2, NP), jnp.float32)]
    scratch = [
        pltpu.VMEM((N,), jnp.float32),
        pltpu.VMEM((N,), jnp.float32),
        pltpu.VMEM((B,), jnp.int32),
        pltpu.VMEM((B,), jnp.int32),
        pltpu.VMEM((B,), jnp.float32),
    ]
    if with_extras:
        out_type += [jax.ShapeDtypeStruct((2, NP), jnp.float32)] * 3
        scratch += [pltpu.VMEM((B,), jnp.float32)]
    scratch += [
        pltpu.VMEM((B,), jnp.float32),
        pltpu.VMEM((B,), jnp.float32),
        pltpu.VMEM((B, 128), jnp.float32),
        pltpu.VMEM_SHARED((NP, 128), jnp.float32),
        pltpu.VMEM_SHARED((NP,), jnp.float32),
    ]
    if with_extras:
        scratch += [pltpu.VMEM_SHARED((NP,), jnp.float32)] * 3
    return pl.kernel(
        functools.partial(_sc_body, with_extras),
        out_type=out_type,
        mesh=mesh,
        scratch_types=scratch,
        compiler_params=pltpu.CompilerParams(needs_layout_passes=False),
    )


def kernel(x, edge_index, edge_attr, W_node, b_node, W_eenc, b_eenc,
           W0, att_src0, att_dst0, We0, att_edge0, bias0, bn_w0, bn_b0,
           W1, att_src1, att_dst1, We1, att_edge1, bias1, bn_w1, bn_b1,
           W_out, b_out):
    src = edge_index[0]
    dst = edge_index[1]

    P0 = W_node @ W0
    bx0 = b_node @ W0
    us0 = P0 @ att_src0
    ud0 = P0 @ att_dst0
    z = jnp.zeros((128, 126), jnp.float32)
    wc0 = jnp.concatenate([P0, us0[:, None], ud0[:, None], z], axis=1)
    bc0 = jnp.concatenate(
        [bx0, jnp.stack([bx0 @ att_src0, bx0 @ att_dst0]),
         jnp.zeros((126,), jnp.float32)])

    we0 = We0 @ att_edge0
    we1 = We1 @ att_edge1
    G = jnp.stack([W_eenc @ we0, W_eenc @ we1], axis=1)
    cvec = jnp.stack([b_eenc @ we0, b_eenc @ we1])

    us1 = W1 @ att_src1
    ud1 = W1 @ att_dst1
    wc1 = jnp.concatenate([W1, us1[:, None], ud1[:, None], z], axis=1)
    bc1 = jnp.zeros((256,), jnp.float32)

    big0 = _mm(x, wc0, bc0, 2000)
    xs0 = big0[:, :128]
    asrc0 = big0[:, 128]
    adst0 = big0[:, 129]

    aer = _mm(edge_attr, G, cvec, 16000)
    aer0 = aer[:, 0]
    aer1 = aer[:, 1]

    num0, den0, deg0, s0p, s1p = _sc_pass(True)(
        xs0, asrc0, adst0, src, dst, aer0, aer1)

    r = lambda v: v.reshape(N, 1)
    big1 = pl.pallas_call(
        _merge_body,
        grid=(5,),
        in_specs=(
            [pl.BlockSpec((2000, 128), lambda i: (i, 0))] * 2
            + [pl.BlockSpec((2000, 1), lambda i: (i, 0))] * 8
            + [pl.BlockSpec((2000, 1), lambda i: (i, 0))] * 2
            + [pl.BlockSpec((2000, 128), lambda i: (i, 0))]
            + [pl.BlockSpec((128, 256), lambda i: (0, 0)),
               pl.BlockSpec((1, 256), lambda i: (0, 0))]
            + [pl.BlockSpec((1, 128), lambda i: (0, 0))] * 3
        ),
        out_specs=pl.BlockSpec((2000, 256), lambda i: (i, 0)),
        out_shape=jax.ShapeDtypeStruct((N, 256), jnp.float32),
    )(num0[0, :N], num0[1, :N],
      r(den0[0, :N]), r(den0[1, :N]), r(deg0[0, :N]), r(deg0[1, :N]),
      r(s0p[0, :N]), r(s0p[1, :N]), r(s1p[0, :N]), r(s1p[1, :N]),
      r(asrc0), r(adst0), xs0, wc1, bc1.reshape(1, 256),
      bias0.reshape(1, 128), bn_w0.reshape(1, 128), bn_b0.reshape(1, 128))

    xs1 = big1[:, :128]
    asrc1 = big1[:, 128]
    adst1 = big1[:, 129]
    pl1 = big1[:, 130]

    num1, den1 = _sc_pass(False)(xs1, asrc1, adst1, src, dst, aer1)

    out = pl.pallas_call(
        _final_body,
        grid=(5,),
        in_specs=(
            [pl.BlockSpec((2000, 128), lambda i: (i, 0))] * 2
            + [pl.BlockSpec((2000, 1), lambda i: (i, 0))] * 3
            + [pl.BlockSpec((2000, 128), lambda i: (i, 0))]
            + [pl.BlockSpec((1, 128), lambda i: (0, 0))] * 3
            + [pl.BlockSpec((128, 128), lambda i: (0, 0)),
               pl.BlockSpec((1, 128), lambda i: (0, 0))]
        ),
        out_specs=pl.BlockSpec((1, 128), lambda i: (0, 0)),
        out_shape=jax.ShapeDtypeStruct((1, 128), jnp.float32),
        scratch_shapes=[pltpu.VMEM((8, 128), jnp.float32)],
    )(num1[0, :N], num1[1, :N],
      r(den1[0, :N]), r(den1[1, :N]), r(pl1),
      xs1, bias1.reshape(1, 128), bn_w1.reshape(1, 128),
      bn_b1.reshape(1, 128), W_out, b_out.reshape(1, 128))

    return out.reshape(-1)


# speedup vs baseline: 23.4732x; 1.3982x over previous
import functools

import jax
import jax.numpy as jnp
from jax import lax
from jax.experimental import pallas as pl
from jax.experimental.pallas import tpu as pltpu
from jax.experimental.pallas import tpu_sc as plsc

N = 10000
E = 320000
NP = 10240
NS = 16
COLS = 64
B = 128
NBW = 160
NBL = (E - (NS - 1) * NBW * B) // B
SUBB = 40
EROWS = NS * NBW
ROWS_PER_SUB = NP // NS
INV_STD = 1.0 / (1.0 + 1e-5) ** 0.5


def _mm_body(x_ref, w_ref, b_ref, o_ref):
    o_ref[...] = (
        jnp.dot(x_ref[...], w_ref[...], preferred_element_type=jnp.float32)
        + b_ref[...]
    )


def _mm(x, w, b, blk):
    m, k = x.shape
    n = w.shape[1]
    return pl.pallas_call(
        _mm_body,
        grid=(m // blk,),
        in_specs=[
            pl.BlockSpec((blk, k), lambda i: (i, 0)),
            pl.BlockSpec((k, n), lambda i: (0, 0)),
            pl.BlockSpec((1, n), lambda i: (0, 0)),
        ],
        out_specs=pl.BlockSpec((blk, n), lambda i: (i, 0)),
        out_shape=jax.ShapeDtypeStruct((m, n), jnp.float32),
    )(x, w, b.reshape(1, n))


def _leaky_exp(a):
    return jnp.exp(jnp.where(a >= 0, a, 0.2 * a))


def _merge_body(numL, numR, den, deg_r, s0_r, s1_r,
                asrc, adst, xs0, wc, bc, bias0, gw, gb, o_ref):
    num = jnp.concatenate([numL[...], numR[...]], axis=1)
    deg = jnp.maximum(deg_r[...], 1.0)
    s0 = s0_r[...]
    s1 = s1_r[...]
    pl0 = _leaky_exp(asrc[...] + adst[...] + s0 / deg)
    out0 = (num + pl0 * xs0[...]) / (den[...] + pl0 + 1e-16) + bias0[...]
    h1 = jnp.maximum(out0 * INV_STD * gw[...] + gb[...], 0.0)
    m = jnp.dot(h1, wc[...], preferred_element_type=jnp.float32) + bc[...]
    a1 = m[:, 128:129] + m[:, 129:130] + s1 / deg
    pl1 = _leaky_exp(a1)
    o_ref[...] = jnp.concatenate([m[:, :130], pl1, m[:, 131:]], axis=1)


def _final_body(numL, numR, den, pl1, xs1, bias1, gw, gb, wo, bo,
                o_ref, acc_ref):
    i = pl.program_id(0)
    num = jnp.concatenate([numL[...], numR[...]], axis=1)
    p = pl1[...]
    out1 = (num + p * xs1[...]) / (den[...] + p + 1e-16) + bias1[...]
    h2 = jnp.maximum(out1 * INV_STD * gw[...] + gb[...], 0.0)
    part = jnp.sum(h2, axis=0, keepdims=True)

    @pl.when(i == 0)
    def _():
        acc_ref[...] = jnp.zeros_like(acc_ref)

    acc_ref[0:1, :] += part

    @pl.when(i == pl.num_programs(0) - 1)
    def _():
        pooled = acc_ref[0:1, :] / float(N)
        o_ref[...] = (
            jnp.dot(pooled, wo[...], preferred_element_type=jnp.float32)
            + bo[...]
        )


def _sc_body(with_extras, *refs):
    if with_extras:
        (xs, asrc_h, adst_h, src_r, dst_r, aer_r, aer2_r,
         num_o, den_o, deg_o, s0_o, s1_o,
         asrc_v, adst_v, src_t, dst_t, aer_t, aer2_t,
         pb, onesb, rows0, rows1, gsem0, gsem1,
         acc_rows, acc_den, acc_deg, acc_s0, acc_s1) = refs
        scalar_accs = (acc_den, acc_deg, acc_s0, acc_s1)
        scalar_outs = (den_o, deg_o, s0_o, s1_o)
    else:
        (xs, asrc_h, adst_h, src_r, dst_r, aer_r,
         num_o, den_o,
         asrc_v, adst_v, src_t, dst_t, aer_t,
         pb, onesb, rows0, rows1, gsem0, gsem1,
         acc_rows, acc_den) = refs
        aer2_r = aer2_t = None
        scalar_accs = (acc_den,)
        scalar_outs = (den_o,)

    c = lax.axis_index("c")
    s = lax.axis_index("s")
    zeros16 = jnp.zeros((16,), jnp.float32)
    ones16 = jnp.ones((16,), jnp.float32)
    on_sc0 = c == 0

    def zrow(r, _):
        for k in range(COLS // 16):
            rows0[r, pl.ds(k * 16, 16)] = zeros16
        return 0
    lax.fori_loop(0, B, zrow, 0)
    for j in range(B // 16):
        onesb[pl.ds(j * 16, 16)] = ones16
        pb[pl.ds(j * 16, 16)] = zeros16

    base = s * ROWS_PER_SUB
    for k in range(ROWS_PER_SUB // B):
        sl = pl.ds(base + k * B, B)
        pltpu.sync_copy(rows0, acc_rows.at[sl])

        @pl.when(on_sc0)
        def _():
            for acc in scalar_accs:
                pltpu.sync_copy(pb, acc.at[sl])

    pltpu.sync_copy(asrc_h, asrc_v)
    pltpu.sync_copy(adst_h, adst_v)
    plsc.subcore_barrier()

    off = c * N
    nb = jnp.where(s == NS - 1, NBL, NBW)
    n_sub = (nb + SUBB - 1) // SUBB

    def do_batch(b, inner_nb, rbuf, gsem, obuf, ogsem):
        for j in range(B // 16):
            sl = pl.ds(j * 16, 16)
            s16 = src_t[b, sl] - off
            d16 = dst_t[b, sl]
            a = (plsc.load_gather(asrc_v, [s16])
                 + plsc.load_gather(adst_v, [d16])
                 + aer_t[b, sl])
            pb[sl] = jnp.exp(jnp.where(a >= 0, a, 0.2 * a))

        pltpu.make_async_copy(xs.at[src_t.at[b]], rbuf, gsem).wait()

        @pl.when(b + 1 < inner_nb)
        def _():
            pltpu.async_copy(xs.at[src_t.at[b + 1]], obuf, ogsem)

        @plsc.parallel_loop(0, B, unroll=4)
        def _(e):
            pe = plsc.load_gather(pb, [jnp.full((16,), e, jnp.int32)])
            for k in range(COLS // 16):
                sl2 = pl.ds(k * 16, 16)
                rbuf[e, sl2] = rbuf[e, sl2] * pe

        idx = dst_t.at[b]
        pltpu.sync_copy(rbuf, acc_rows.at[idx], add=True)

        @pl.when(on_sc0)
        def _():
            pltpu.sync_copy(pb, acc_den.at[idx], add=True)
            if with_extras:
                pltpu.sync_copy(onesb, acc_deg.at[idx], add=True)
                pltpu.sync_copy(aer_t.at[b], acc_s0.at[idx], add=True)
                pltpu.sync_copy(aer2_t.at[b], acc_s1.at[idx], add=True)

    def sub_chunk(sc_i, _):
        crow = s * NBW + sc_i * SUBB
        pltpu.sync_copy(src_r.at[pl.ds(crow, SUBB)], src_t)
        pltpu.sync_copy(dst_r.at[pl.ds(crow, SUBB)], dst_t)
        pltpu.sync_copy(aer_r.at[pl.ds(crow, SUBB)], aer_t)
        if with_extras:
            pltpu.sync_copy(aer2_r.at[pl.ds(crow, SUBB)], aer2_t)

        def offs(r, _):
            for j in range(B // 16):
                sl = pl.ds(j * 16, 16)
                src_t[r, sl] = src_t[r, sl] + off
            return 0
        lax.fori_loop(0, SUBB, offs, 0)

        inner_nb = jnp.minimum(SUBB, nb - sc_i * SUBB)
        pltpu.async_copy(xs.at[src_t.at[0]], rows0, gsem0)

        def body(b, _):
            @pl.when(b % 2 == 0)
            def _():
                do_batch(b, inner_nb, rows0, gsem0, rows1, gsem1)

            @pl.when(b % 2 == 1)
            def _():
                do_batch(b, inner_nb, rows1, gsem1, rows0, gsem0)
            return 0
        lax.fori_loop(0, inner_nb, body, 0)
        return 0
    lax.fori_loop(0, n_sub, sub_chunk, 0)
    plsc.subcore_barrier()

    for k in range(ROWS_PER_SUB // B):
        sl = pl.ds(base + k * B, B)
        pltpu.sync_copy(acc_rows.at[sl], num_o.at[c, sl])

        @pl.when(on_sc0)
        def _():
            for acc, out in zip(scalar_accs, scalar_outs):
                pltpu.sync_copy(acc.at[sl], out.at[sl])


def _sc_pass(with_extras):
    mesh = plsc.VectorSubcoreMesh(core_axis_name="c", subcore_axis_name="s")
    out_type = [jax.ShapeDtypeStruct((2, NP, COLS), jnp.float32),
                jax.ShapeDtypeStruct((NP,), jnp.float32)]
    scratch = [
        pltpu.VMEM((N,), jnp.float32),
        pltpu.VMEM((N,), jnp.float32),
        pltpu.VMEM((SUBB, B), jnp.int32),
        pltpu.VMEM((SUBB, B), jnp.int32),
        pltpu.VMEM((SUBB, B), jnp.float32),
    ]
    if with_extras:
        out_type += [jax.ShapeDtypeStruct((NP,), jnp.float32)] * 3
        scratch += [pltpu.VMEM((SUBB, B), jnp.float32)]
    scratch += [
        pltpu.VMEM((B,), jnp.float32),
        pltpu.VMEM((B,), jnp.float32),
        pltpu.VMEM((B, COLS), jnp.float32),
        pltpu.VMEM((B, COLS), jnp.float32),
        pltpu.SemaphoreType.DMA,
        pltpu.SemaphoreType.DMA,
        pltpu.VMEM_SHARED((NP, COLS), jnp.float32),
        pltpu.VMEM_SHARED((NP,), jnp.float32),
    ]
    if with_extras:
        scratch += [pltpu.VMEM_SHARED((NP,), jnp.float32)] * 3
    return pl.kernel(
        functools.partial(_sc_body, with_extras),
        out_type=out_type,
        mesh=mesh,
        scratch_types=scratch,
        compiler_params=pltpu.CompilerParams(
            needs_layout_passes=False, use_tc_tiling_on_sc=False),
    )


def kernel(x, edge_index, edge_attr, W_node, b_node, W_eenc, b_eenc,
           W0, att_src0, att_dst0, We0, att_edge0, bias0, bn_w0, bn_b0,
           W1, att_src1, att_dst1, We1, att_edge1, bias1, bn_w1, bn_b1,
           W_out, b_out):
    src = edge_index[0]
    dst = edge_index[1]

    P0 = W_node @ W0
    bx0 = b_node @ W0
    us0 = P0 @ att_src0
    ud0 = P0 @ att_dst0
    z = jnp.zeros((128, 126), jnp.float32)
    wc0 = jnp.concatenate([P0, us0[:, None], ud0[:, None], z], axis=1)
    bc0 = jnp.concatenate(
        [bx0, jnp.stack([bx0 @ att_src0, bx0 @ att_dst0]),
         jnp.zeros((126,), jnp.float32)])

    we0 = We0 @ att_edge0
    we1 = We1 @ att_edge1
    G = jnp.stack([W_eenc @ we0, W_eenc @ we1], axis=1)
    cvec = jnp.stack([b_eenc @ we0, b_eenc @ we1])

    us1 = W1 @ att_src1
    ud1 = W1 @ att_dst1
    wc1 = jnp.concatenate([W1, us1[:, None], ud1[:, None], z], axis=1)
    bc1 = jnp.zeros((256,), jnp.float32)

    big0 = _mm(x, wc0, bc0, 2000)
    xs0 = big0[:, :128]
    asrc0 = big0[:, 128]
    adst0 = big0[:, 129]

    aer = _mm(edge_attr, G, cvec, 16000)
    pad = ((0, EROWS - E // B), (0, 0))
    src_r = jnp.pad(src.reshape(E // B, B), pad)
    dst_r = jnp.pad(dst.reshape(E // B, B), pad)
    aer0_r = jnp.pad(aer[:, 0].reshape(E // B, B), pad)
    aer1_r = jnp.pad(aer[:, 1].reshape(E // B, B), pad)

    xs0f = jnp.concatenate([xs0[:, :COLS], xs0[:, COLS:]], axis=0)
    num0, den0, deg0, s0p, s1p = _sc_pass(True)(
        xs0f, asrc0, adst0, src_r, dst_r, aer0_r, aer1_r)

    r = lambda v: v.reshape(N, 1)
    big1 = pl.pallas_call(
        _merge_body,
        grid=(5,),
        in_specs=(
            [pl.BlockSpec((2000, COLS), lambda i: (i, 0))] * 2
            + [pl.BlockSpec((2000, 1), lambda i: (i, 0))] * 4
            + [pl.BlockSpec((2000, 1), lambda i: (i, 0))] * 2
            + [pl.BlockSpec((2000, 128), lambda i: (i, 0))]
            + [pl.BlockSpec((128, 256), lambda i: (0, 0)),
               pl.BlockSpec((1, 256), lambda i: (0, 0))]
            + [pl.BlockSpec((1, 128), lambda i: (0, 0))] * 3
        ),
        out_specs=pl.BlockSpec((2000, 256), lambda i: (i, 0)),
        out_shape=jax.ShapeDtypeStruct((N, 256), jnp.float32),
    )(num0[0, :N], num0[1, :N],
      r(den0[:N]), r(deg0[:N]), r(s0p[:N]), r(s1p[:N]),
      r(asrc0), r(adst0), xs0, wc1, bc1.reshape(1, 256),
      bias0.reshape(1, 128), bn_w0.reshape(1, 128), bn_b0.reshape(1, 128))

    xs1 = big1[:, :128]
    asrc1 = big1[:, 128]
    adst1 = big1[:, 129]
    pl1 = big1[:, 130]

    xs1f = jnp.concatenate([xs1[:, :COLS], xs1[:, COLS:]], axis=0)
    num1, den1 = _sc_pass(False)(xs1f, asrc1, adst1, src_r, dst_r, aer1_r)

    out = pl.pallas_call(
        _final_body,
        grid=(5,),
        in_specs=(
            [pl.BlockSpec((2000, COLS), lambda i: (i, 0))] * 2
            + [pl.BlockSpec((2000, 1), lambda i: (i, 0))] * 2
            + [pl.BlockSpec((2000, 128), lambda i: (i, 0))]
            + [pl.BlockSpec((1, 128), lambda i: (0, 0))] * 3
            + [pl.BlockSpec((128, 128), lambda i: (0, 0)),
               pl.BlockSpec((1, 128), lambda i: (0, 0))]
        ),
        out_specs=pl.BlockSpec((1, 128), lambda i: (0, 0)),
        out_shape=jax.ShapeDtypeStruct((1, 128), jnp.float32),
        scratch_shapes=[pltpu.VMEM((8, 128), jnp.float32)],
    )(num1[0, :N], num1[1, :N],
      r(den1[:N]), r(pl1),
      xs1, bias1.reshape(1, 128), bn_w1.reshape(1, 128),
      bn_b1.reshape(1, 128), W_out, b_out.reshape(1, 128))

    return out.reshape(-1)


# speedup vs baseline: 26.0236x; 1.1087x over previous
import functools

import jax
import jax.numpy as jnp
from jax import lax
from jax.experimental import pallas as pl
from jax.experimental.pallas import tpu as pltpu
from jax.experimental.pallas import tpu_sc as plsc

N = 10000
E = 320000
NP = 10240
NS = 16
COLS = 64
B = 128
NBW = 160
NBL = (E - (NS - 1) * NBW * B) // B
SUBB = 40
EROWS = NS * NBW
ROWS_PER_SUB = NP // NS
INV_STD = 1.0 / (1.0 + 1e-5) ** 0.5


def _mm_body(x_ref, w_ref, b_ref, o_ref):
    o_ref[...] = (
        jnp.dot(x_ref[...], w_ref[...], preferred_element_type=jnp.float32)
        + b_ref[...]
    )


def _mm(x, w, b, blk):
    m, k = x.shape
    n = w.shape[1]
    return pl.pallas_call(
        _mm_body,
        grid=(m // blk,),
        in_specs=[
            pl.BlockSpec((blk, k), lambda i: (i, 0)),
            pl.BlockSpec((k, n), lambda i: (0, 0)),
            pl.BlockSpec((1, n), lambda i: (0, 0)),
        ],
        out_specs=pl.BlockSpec((blk, n), lambda i: (i, 0)),
        out_shape=jax.ShapeDtypeStruct((m, n), jnp.float32),
    )(x, w, b.reshape(1, n))


def _leaky_exp(a):
    return jnp.exp(jnp.where(a >= 0, a, 0.2 * a))


def _merge_body(numL, numR, den, deg_r, s0_r, s1_r,
                asrc, adst, xs0, wc, bc, bias0, gw, gb, o_ref):
    num = jnp.concatenate([numL[0], numR[0]], axis=1)
    deg = jnp.maximum(deg_r[...], 1.0)
    s0 = s0_r[...]
    s1 = s1_r[...]
    pl0 = _leaky_exp(asrc[...] + adst[...] + s0 / deg)
    out0 = (num + pl0 * xs0[...]) / (den[...] + pl0 + 1e-16) + bias0[...]
    h1 = jnp.maximum(out0 * INV_STD * gw[...] + gb[...], 0.0)
    m = jnp.dot(h1, wc[...], preferred_element_type=jnp.float32) + bc[...]
    a1 = m[:, 128:129] + m[:, 129:130] + s1 / deg
    pl1 = _leaky_exp(a1)
    o_ref[...] = jnp.concatenate([m[:, :130], pl1, m[:, 131:]], axis=1)


def _final_body(numL, numR, den, pl1, xs1, bias1, gw, gb, wo, bo,
                o_ref, acc_ref):
    i = pl.program_id(0)
    num = jnp.concatenate([numL[0], numR[0]], axis=1)
    p = pl1[...]
    out1 = (num + p * xs1[...]) / (den[...] + p + 1e-16) + bias1[...]
    h2 = jnp.maximum(out1 * INV_STD * gw[...] + gb[...], 0.0)
    part = jnp.sum(h2, axis=0, keepdims=True)

    @pl.when(i == 0)
    def _():
        acc_ref[...] = jnp.zeros_like(acc_ref)

    acc_ref[0:1, :] += part

    @pl.when(i == pl.num_programs(0) - 1)
    def _():
        pooled = acc_ref[0:1, :] / float(N)
        o_ref[...] = (
            jnp.dot(pooled, wo[...], preferred_element_type=jnp.float32)
            + bo[...]
        )


def _sc_body(with_extras, *refs):
    if with_extras:
        (xs, asrc_h, adst_h, src_r, dst_r, aer_r, aer2_r,
         num_o, den_o, deg_o, s0_o, s1_o,
         asrc_v, adst_v, src_t, dst_t, aer_t, aer2_t,
         pb0, pb1, onesb, rows0, rows1,
         gsem0, gsem1, ssem0, ssem1,
         acc_rows, acc_den, acc_deg, acc_s0, acc_s1) = refs
    else:
        (xs, asrc_h, adst_h, src_r, dst_r, aer_r,
         num_o, den_o,
         asrc_v, adst_v, src_t, dst_t, aer_t,
         pb0, pb1, onesb, rows0, rows1,
         gsem0, gsem1, ssem0, ssem1,
         acc_rows, acc_den) = refs
        aer2_r = aer2_t = None
        acc_deg = acc_s0 = acc_s1 = None
        deg_o = s0_o = s1_o = None

    c = lax.axis_index("c")
    s = lax.axis_index("s")
    zeros16 = jnp.zeros((16,), jnp.float32)
    ones16 = jnp.ones((16,), jnp.float32)
    on_sc0 = c == 0
    on_sc1 = c == 1
    if with_extras:
        my_scalars_sc0 = ((acc_den, den_o), (acc_deg, deg_o))
        my_scalars_sc1 = ((acc_s0, s0_o), (acc_s1, s1_o))
    else:
        my_scalars_sc0 = ((acc_den, den_o),)
        my_scalars_sc1 = ()

    def zrow(r, _):
        for k in range(COLS // 16):
            rows0[r, pl.ds(k * 16, 16)] = zeros16
        return 0
    lax.fori_loop(0, B, zrow, 0)
    for j in range(B // 16):
        onesb[pl.ds(j * 16, 16)] = ones16
        pb0[pl.ds(j * 16, 16)] = zeros16

    base = s * ROWS_PER_SUB
    for k in range(ROWS_PER_SUB // B):
        sl = pl.ds(base + k * B, B)
        pltpu.sync_copy(rows0, acc_rows.at[sl])

        @pl.when(on_sc0)
        def _():
            for acc, _o in my_scalars_sc0:
                pltpu.sync_copy(pb0, acc.at[sl])

        if my_scalars_sc1:
            @pl.when(on_sc1)
            def _():
                for acc, _o in my_scalars_sc1:
                    pltpu.sync_copy(pb0, acc.at[sl])

    pltpu.sync_copy(asrc_h, asrc_v)
    pltpu.sync_copy(adst_h, adst_v)
    plsc.subcore_barrier()

    off = c * N
    nb = jnp.where(s == NS - 1, NBL, NBW)
    n_sub = (nb + SUBB - 1) // SUBB

    def drain_scatters(orbuf, opbuf, ossem):
        idx0 = dst_t.at[0]
        pltpu.make_async_copy(orbuf, acc_rows.at[idx0], ossem).wait()

        @pl.when(on_sc0)
        def _():
            pltpu.make_async_copy(opbuf, acc_den.at[idx0], ossem).wait()
            if with_extras:
                pltpu.make_async_copy(onesb, acc_deg.at[idx0], ossem).wait()

        if with_extras:
            @pl.when(on_sc1)
            def _():
                pltpu.make_async_copy(aer_t.at[0], acc_s0.at[idx0],
                                      ossem).wait()
                pltpu.make_async_copy(aer2_t.at[0], acc_s1.at[idx0],
                                      ossem).wait()

    def do_batch(b, inner_nb, rbuf, pbuf, gsem, ssem,
                 obuf, opbuf, ogsem, ossem):
        for j in range(B // 16):
            sl = pl.ds(j * 16, 16)
            s16 = src_t[b, sl] - off
            d16 = dst_t[b, sl]
            a = (plsc.load_gather(asrc_v, [s16])
                 + plsc.load_gather(adst_v, [d16])
                 + aer_t[b, sl])
            pbuf[sl] = jnp.exp(jnp.where(a >= 0, a, 0.2 * a))

        pltpu.make_async_copy(xs.at[src_t.at[b]], rbuf, gsem).wait()

        @pl.when(b > 0)
        def _():
            drain_scatters(obuf, opbuf, ossem)

        @pl.when(b + 1 < inner_nb)
        def _():
            pltpu.async_copy(xs.at[src_t.at[b + 1]], obuf, ogsem)

        @plsc.parallel_loop(0, B, unroll=8)
        def _(e):
            pe = plsc.load_gather(pbuf, [jnp.full((16,), e, jnp.int32)])
            for k in range(COLS // 16):
                sl2 = pl.ds(k * 16, 16)
                rbuf[e, sl2] = rbuf[e, sl2] * pe

        idx = dst_t.at[b]
        pltpu.async_copy(rbuf, acc_rows.at[idx], ssem, add=True)

        @pl.when(on_sc0)
        def _():
            pltpu.async_copy(pbuf, acc_den.at[idx], ssem, add=True)
            if with_extras:
                pltpu.async_copy(onesb, acc_deg.at[idx], ssem, add=True)

        if with_extras:
            @pl.when(on_sc1)
            def _():
                pltpu.async_copy(aer_t.at[b], acc_s0.at[idx], ssem, add=True)
                pltpu.async_copy(aer2_t.at[b], acc_s1.at[idx], ssem, add=True)

    def sub_chunk(sc_i, _):
        crow = s * NBW + sc_i * SUBB
        pltpu.sync_copy(src_r.at[pl.ds(crow, SUBB)], src_t)
        pltpu.sync_copy(dst_r.at[pl.ds(crow, SUBB)], dst_t)
        pltpu.sync_copy(aer_r.at[pl.ds(crow, SUBB)], aer_t)
        if with_extras:
            pltpu.sync_copy(aer2_r.at[pl.ds(crow, SUBB)], aer2_t)

        def offs(r, _):
            for j in range(B // 16):
                sl = pl.ds(j * 16, 16)
                src_t[r, sl] = src_t[r, sl] + off
            return 0
        lax.fori_loop(0, SUBB, offs, 0)

        inner_nb = jnp.minimum(SUBB, nb - sc_i * SUBB)
        pltpu.async_copy(xs.at[src_t.at[0]], rows0, gsem0)

        def body(b, _):
            @pl.when(b % 2 == 0)
            def _():
                do_batch(b, inner_nb, rows0, pb0, gsem0, ssem0,
                         rows1, pb1, gsem1, ssem1)

            @pl.when(b % 2 == 1)
            def _():
                do_batch(b, inner_nb, rows1, pb1, gsem1, ssem1,
                         rows0, pb0, gsem0, ssem0)
            return 0
        lax.fori_loop(0, inner_nb, body, 0)
        drain_scatters(rows1, pb1, ssem1)
        return 0
    lax.fori_loop(0, n_sub, sub_chunk, 0)
    plsc.subcore_barrier()

    for k in range(ROWS_PER_SUB // B):
        sl = pl.ds(base + k * B, B)
        pltpu.sync_copy(acc_rows.at[sl], num_o.at[c, sl])

        @pl.when(on_sc0)
        def _():
            for acc, out in my_scalars_sc0:
                pltpu.sync_copy(acc.at[sl], out.at[sl])

        if my_scalars_sc1:
            @pl.when(on_sc1)
            def _():
                for acc, out in my_scalars_sc1:
                    pltpu.sync_copy(acc.at[sl], out.at[sl])


def _sc_pass(with_extras):
    mesh = plsc.VectorSubcoreMesh(core_axis_name="c", subcore_axis_name="s")
    out_type = [jax.ShapeDtypeStruct((2, NP, COLS), jnp.float32),
                jax.ShapeDtypeStruct((NP,), jnp.float32)]
    scratch = [
        pltpu.VMEM((N,), jnp.float32),
        pltpu.VMEM((N,), jnp.float32),
        pltpu.VMEM((SUBB, B), jnp.int32),
        pltpu.VMEM((SUBB, B), jnp.int32),
        pltpu.VMEM((SUBB, B), jnp.float32),
    ]
    if with_extras:
        out_type += [jax.ShapeDtypeStruct((NP,), jnp.float32)] * 3
        scratch += [pltpu.VMEM((SUBB, B), jnp.float32)]
    scratch += [
        pltpu.VMEM((B,), jnp.float32),
        pltpu.VMEM((B,), jnp.float32),
        pltpu.VMEM((B,), jnp.float32),
        pltpu.VMEM((B, COLS), jnp.float32),
        pltpu.VMEM((B, COLS), jnp.float32),
        pltpu.SemaphoreType.DMA,
        pltpu.SemaphoreType.DMA,
        pltpu.SemaphoreType.DMA,
        pltpu.SemaphoreType.DMA,
        pltpu.VMEM_SHARED((NP, COLS), jnp.float32),
        pltpu.VMEM_SHARED((NP,), jnp.float32),
    ]
    if with_extras:
        scratch += [pltpu.VMEM_SHARED((NP,), jnp.float32)] * 3
    return pl.kernel(
        functools.partial(_sc_body, with_extras),
        out_type=out_type,
        mesh=mesh,
        scratch_types=scratch,
        compiler_params=pltpu.CompilerParams(
            needs_layout_passes=False, use_tc_tiling_on_sc=False),
    )


def kernel(x, edge_index, edge_attr, W_node, b_node, W_eenc, b_eenc,
           W0, att_src0, att_dst0, We0, att_edge0, bias0, bn_w0, bn_b0,
           W1, att_src1, att_dst1, We1, att_edge1, bias1, bn_w1, bn_b1,
           W_out, b_out):
    src = edge_index[0]
    dst = edge_index[1]

    P0 = W_node @ W0
    bx0 = b_node @ W0
    us0 = P0 @ att_src0
    ud0 = P0 @ att_dst0
    z = jnp.zeros((128, 126), jnp.float32)
    wc0 = jnp.concatenate([P0, us0[:, None], ud0[:, None], z], axis=1)
    bc0 = jnp.concatenate(
        [bx0, jnp.stack([bx0 @ att_src0, bx0 @ att_dst0]),
         jnp.zeros((126,), jnp.float32)])

    we0 = We0 @ att_edge0
    we1 = We1 @ att_edge1
    G = jnp.stack([W_eenc @ we0, W_eenc @ we1], axis=1)
    cvec = jnp.stack([b_eenc @ we0, b_eenc @ we1])

    us1 = W1 @ att_src1
    ud1 = W1 @ att_dst1
    wc1 = jnp.concatenate([W1, us1[:, None], ud1[:, None], z], axis=1)
    bc1 = jnp.zeros((256,), jnp.float32)

    big0 = _mm(x, wc0, bc0, 2000)
    xs0 = big0[:, :128]
    asrc0 = big0[:, 128]
    adst0 = big0[:, 129]

    aer = _mm(edge_attr, G, cvec, 16000)
    pad = ((0, EROWS - E // B), (0, 0))
    src_r = jnp.pad(src.reshape(E // B, B), pad)
    dst_r = jnp.pad(dst.reshape(E // B, B), pad)
    aer0_r = jnp.pad(aer[:, 0].reshape(E // B, B), pad)
    aer1_r = jnp.pad(aer[:, 1].reshape(E // B, B), pad)

    xs0f = jnp.concatenate([xs0[:, :COLS], xs0[:, COLS:]], axis=0)
    num0, den0, deg0, s0p, s1p = _sc_pass(True)(
        xs0f, asrc0, adst0, src_r, dst_r, aer0_r, aer1_r)

    r = lambda v: v.reshape(N, 1)
    rp = lambda v: v.reshape(NP, 1)
    big1 = pl.pallas_call(
        _merge_body,
        grid=(5,),
        in_specs=(
            [pl.BlockSpec((1, 2000, COLS), lambda i: (0, i, 0)),
             pl.BlockSpec((1, 2000, COLS), lambda i: (1, i, 0))]
            + [pl.BlockSpec((2000, 1), lambda i: (i, 0))] * 4
            + [pl.BlockSpec((2000, 1), lambda i: (i, 0))] * 2
            + [pl.BlockSpec((2000, 128), lambda i: (i, 0))]
            + [pl.BlockSpec((128, 256), lambda i: (0, 0)),
               pl.BlockSpec((1, 256), lambda i: (0, 0))]
            + [pl.BlockSpec((1, 128), lambda i: (0, 0))] * 3
        ),
        out_specs=pl.BlockSpec((2000, 256), lambda i: (i, 0)),
        out_shape=jax.ShapeDtypeStruct((N, 256), jnp.float32),
    )(num0, num0,
      rp(den0), rp(deg0), rp(s0p), rp(s1p),
      r(asrc0), r(adst0), xs0, wc1, bc1.reshape(1, 256),
      bias0.reshape(1, 128), bn_w0.reshape(1, 128), bn_b0.reshape(1, 128))

    xs1 = big1[:, :128]
    asrc1 = big1[:, 128]
    adst1 = big1[:, 129]
    pl1 = big1[:, 130]

    xs1f = jnp.concatenate([xs1[:, :COLS], xs1[:, COLS:]], axis=0)
    num1, den1 = _sc_pass(False)(xs1f, asrc1, adst1, src_r, dst_r, aer1_r)

    out = pl.pallas_call(
        _final_body,
        grid=(5,),
        in_specs=(
            [pl.BlockSpec((1, 2000, COLS), lambda i: (0, i, 0)),
             pl.BlockSpec((1, 2000, COLS), lambda i: (1, i, 0))]
            + [pl.BlockSpec((2000, 1), lambda i: (i, 0))] * 2
            + [pl.BlockSpec((2000, 128), lambda i: (i, 0))]
            + [pl.BlockSpec((1, 128), lambda i: (0, 0))] * 3
            + [pl.BlockSpec((128, 128), lambda i: (0, 0)),
               pl.BlockSpec((1, 128), lambda i: (0, 0))]
        ),
        out_specs=pl.BlockSpec((1, 128), lambda i: (0, 0)),
        out_shape=jax.ShapeDtypeStruct((1, 128), jnp.float32),
        scratch_shapes=[pltpu.VMEM((8, 128), jnp.float32)],
    )(num1, num1,
      rp(den1), r(pl1),
      xs1, bias1.reshape(1, 128), bn_w1.reshape(1, 128),
      bn_b1.reshape(1, 128), W_out, b_out.reshape(1, 128))

    return out.reshape(-1)


# speedup vs baseline: 31.9076x; 1.2261x over previous
import functools

import jax
import jax.numpy as jnp
from jax import lax
from jax.experimental import pallas as pl
from jax.experimental.pallas import tpu as pltpu
from jax.experimental.pallas import tpu_sc as plsc

N = 10000
E = 320000
NP = 10240
NS = 16
COLS = 64
B = 128
NBW = 160
NBL = (E - (NS - 1) * NBW * B) // B
SUBB = 40
EROWS = NS * NBW
ROWS_PER_SUB = NP // NS
INV_STD = 1.0 / (1.0 + 1e-5) ** 0.5


def _mm_body(x_ref, w_ref, b_ref, o_ref):
    o_ref[...] = (
        jnp.dot(x_ref[...], w_ref[...], preferred_element_type=jnp.float32)
        + b_ref[...]
    )


def _aer_body(ea, gt0_ref, gt1_ref, s_ref, c0_ref, c1_ref, o0, o1):
    x = ea[...]
    smat = s_ref[...]
    o0[...] = jnp.dot(x * gt0_ref[...], smat,
                      preferred_element_type=jnp.float32) + c0_ref[...]
    o1[...] = jnp.dot(x * gt1_ref[...], smat,
                      preferred_element_type=jnp.float32) + c1_ref[...]


def _mm(x, w, b, blk):
    m, k = x.shape
    n = w.shape[1]
    return pl.pallas_call(
        _mm_body,
        grid=(m // blk,),
        in_specs=[
            pl.BlockSpec((blk, k), lambda i: (i, 0)),
            pl.BlockSpec((k, n), lambda i: (0, 0)),
            pl.BlockSpec((1, n), lambda i: (0, 0)),
        ],
        out_specs=pl.BlockSpec((blk, n), lambda i: (i, 0)),
        out_shape=jax.ShapeDtypeStruct((m, n), jnp.float32),
    )(x, w, b.reshape(1, n))


def _leaky_exp(a):
    return jnp.exp(jnp.where(a >= 0, a, 0.2 * a))


def _merge_body(numL, numR, den, deg_r, s0_r, s1_r,
                asrc, adst, xs0, wc, bc, bias0, gw, gb, o_ref):
    num = jnp.concatenate([numL[0], numR[0]], axis=1)
    deg = jnp.maximum(deg_r[...], 1.0)
    s0 = s0_r[...]
    s1 = s1_r[...]
    pl0 = _leaky_exp(asrc[...] + adst[...] + s0 / deg)
    out0 = (num + pl0 * xs0[...]) / (den[...] + pl0 + 1e-16) + bias0[...]
    h1 = jnp.maximum(out0 * INV_STD * gw[...] + gb[...], 0.0)
    m = jnp.dot(h1, wc[...], preferred_element_type=jnp.float32) + bc[...]
    a1 = m[:, 128:129] + m[:, 129:130] + s1 / deg
    pl1 = _leaky_exp(a1)
    o_ref[...] = jnp.concatenate([m[:, :130], pl1, m[:, 131:]], axis=1)


def _final_body(numL, numR, den, pl1, xs1, bias1, gw, gb, wo, bo,
                o_ref, acc_ref):
    i = pl.program_id(0)
    num = jnp.concatenate([numL[0], numR[0]], axis=1)
    p = pl1[...]
    out1 = (num + p * xs1[...]) / (den[...] + p + 1e-16) + bias1[...]
    h2 = jnp.maximum(out1 * INV_STD * gw[...] + gb[...], 0.0)
    part = jnp.sum(h2, axis=0, keepdims=True)

    @pl.when(i == 0)
    def _():
        acc_ref[...] = jnp.zeros_like(acc_ref)

    acc_ref[0:1, :] += part

    @pl.when(i == pl.num_programs(0) - 1)
    def _():
        pooled = acc_ref[0:1, :] / float(N)
        o_ref[...] = (
            jnp.dot(pooled, wo[...], preferred_element_type=jnp.float32)
            + bo[...]
        )


def _sc_body(with_extras, *refs):
    if with_extras:
        (xs, asrc_h, adst_h, src_r, dst_r, aer_r, aer2_r,
         num_o, den_o, deg_o, s0_o, s1_o,
         asrc_v, adst_v, src_t, dst_t, aer_t, aer2_t,
         pb0, pb1, onesb, rows0, rows1,
         gsem0, gsem1, ssem0, ssem1,
         acc_rows, acc_den, acc_deg, acc_s0, acc_s1) = refs
    else:
        (xs, asrc_h, adst_h, src_r, dst_r, aer_r,
         num_o, den_o,
         asrc_v, adst_v, src_t, dst_t, aer_t,
         pb0, pb1, onesb, rows0, rows1,
         gsem0, gsem1, ssem0, ssem1,
         acc_rows, acc_den) = refs
        aer2_r = aer2_t = None
        acc_deg = acc_s0 = acc_s1 = None
        deg_o = s0_o = s1_o = None

    c = lax.axis_index("c")
    s = lax.axis_index("s")
    zeros16 = jnp.zeros((16,), jnp.float32)
    ones16 = jnp.ones((16,), jnp.float32)
    on_sc0 = c == 0
    on_sc1 = c == 1
    if with_extras:
        my_scalars_sc0 = ((acc_den, den_o), (acc_deg, deg_o))
        my_scalars_sc1 = ((acc_s0, s0_o), (acc_s1, s1_o))
    else:
        my_scalars_sc0 = ((acc_den, den_o),)
        my_scalars_sc1 = ()

    def zrow(r, _):
        for k in range(COLS // 16):
            rows0[r, pl.ds(k * 16, 16)] = zeros16
        return 0
    lax.fori_loop(0, B, zrow, 0)
    for j in range(B // 16):
        onesb[pl.ds(j * 16, 16)] = ones16
        pb0[pl.ds(j * 16, 16)] = zeros16

    base = s * ROWS_PER_SUB
    for k in range(ROWS_PER_SUB // B):
        sl = pl.ds(base + k * B, B)
        pltpu.sync_copy(rows0, acc_rows.at[sl])

        @pl.when(on_sc0)
        def _():
            for acc, _o in my_scalars_sc0:
                pltpu.sync_copy(pb0, acc.at[sl])

        if my_scalars_sc1:
            @pl.when(on_sc1)
            def _():
                for acc, _o in my_scalars_sc1:
                    pltpu.sync_copy(pb0, acc.at[sl])

    pltpu.sync_copy(asrc_h, asrc_v)
    pltpu.sync_copy(adst_h, adst_v)
    plsc.subcore_barrier()

    off = c * N
    nb = jnp.where(s == NS - 1, NBL, NBW)
    n_sub = (nb + SUBB - 1) // SUBB

    def drain_scatters(orbuf, opbuf, ossem):
        idx0 = dst_t.at[0]
        pltpu.make_async_copy(orbuf, acc_rows.at[idx0], ossem).wait()

        @pl.when(on_sc0)
        def _():
            pltpu.make_async_copy(opbuf, acc_den.at[idx0], ossem).wait()
            if with_extras:
                pltpu.make_async_copy(onesb, acc_deg.at[idx0], ossem).wait()

        if with_extras:
            @pl.when(on_sc1)
            def _():
                pltpu.make_async_copy(aer_t.at[0], acc_s0.at[idx0],
                                      ossem).wait()
                pltpu.make_async_copy(aer2_t.at[0], acc_s1.at[idx0],
                                      ossem).wait()

    def do_batch(b, inner_nb, rbuf, pbuf, gsem, ssem,
                 obuf, opbuf, ogsem, ossem):
        for j in range(B // 16):
            sl = pl.ds(j * 16, 16)
            s16 = src_t[b, sl] - off
            d16 = dst_t[b, sl]
            a = (plsc.load_gather(asrc_v, [s16])
                 + plsc.load_gather(adst_v, [d16])
                 + aer_t[b, sl])
            pbuf[sl] = jnp.exp(jnp.where(a >= 0, a, 0.2 * a))

        pltpu.make_async_copy(xs.at[src_t.at[b]], rbuf, gsem).wait()

        @pl.when(b > 0)
        def _():
            drain_scatters(obuf, opbuf, ossem)

        @pl.when(b + 1 < inner_nb)
        def _():
            pltpu.async_copy(xs.at[src_t.at[b + 1]], obuf, ogsem)

        @plsc.parallel_loop(0, B, unroll=8)
        def _(e):
            pe = plsc.load_gather(pbuf, [jnp.full((16,), e, jnp.int32)])
            for k in range(COLS // 16):
                sl2 = pl.ds(k * 16, 16)
                rbuf[e, sl2] = rbuf[e, sl2] * pe

        idx = dst_t.at[b]
        pltpu.async_copy(rbuf, acc_rows.at[idx], ssem, add=True)

        @pl.when(on_sc0)
        def _():
            pltpu.async_copy(pbuf, acc_den.at[idx], ssem, add=True)
            if with_extras:
                pltpu.async_copy(onesb, acc_deg.at[idx], ssem, add=True)

        if with_extras:
            @pl.when(on_sc1)
            def _():
                pltpu.async_copy(aer_t.at[b], acc_s0.at[idx], ssem, add=True)
                pltpu.async_copy(aer2_t.at[b], acc_s1.at[idx], ssem, add=True)

    def sub_chunk(sc_i, _):
        crow = s * NBW + sc_i * SUBB
        pltpu.sync_copy(src_r.at[pl.ds(crow, SUBB)], src_t)
        pltpu.sync_copy(dst_r.at[pl.ds(crow, SUBB)], dst_t)
        pltpu.sync_copy(aer_r.at[pl.ds(crow, SUBB)], aer_t)
        if with_extras:
            pltpu.sync_copy(aer2_r.at[pl.ds(crow, SUBB)], aer2_t)

        def offs(r, _):
            for j in range(B // 16):
                sl = pl.ds(j * 16, 16)
                src_t[r, sl] = src_t[r, sl] + off
            return 0
        lax.fori_loop(0, SUBB, offs, 0)

        inner_nb = jnp.minimum(SUBB, nb - sc_i * SUBB)
        pltpu.async_copy(xs.at[src_t.at[0]], rows0, gsem0)

        def body(b, _):
            @pl.when(b % 2 == 0)
            def _():
                do_batch(b, inner_nb, rows0, pb0, gsem0, ssem0,
                         rows1, pb1, gsem1, ssem1)

            @pl.when(b % 2 == 1)
            def _():
                do_batch(b, inner_nb, rows1, pb1, gsem1, ssem1,
                         rows0, pb0, gsem0, ssem0)
            return 0
        lax.fori_loop(0, inner_nb, body, 0)
        drain_scatters(rows1, pb1, ssem1)
        return 0
    lax.fori_loop(0, n_sub, sub_chunk, 0)
    plsc.subcore_barrier()

    for k in range(ROWS_PER_SUB // B):
        sl = pl.ds(base + k * B, B)
        pltpu.sync_copy(acc_rows.at[sl], num_o.at[c, sl])

        @pl.when(on_sc0)
        def _():
            for acc, out in my_scalars_sc0:
                pltpu.sync_copy(acc.at[sl], out.at[sl])

        if my_scalars_sc1:
            @pl.when(on_sc1)
            def _():
                for acc, out in my_scalars_sc1:
                    pltpu.sync_copy(acc.at[sl], out.at[sl])


def _sc_pass(with_extras):
    mesh = plsc.VectorSubcoreMesh(core_axis_name="c", subcore_axis_name="s")
    out_type = [jax.ShapeDtypeStruct((2, NP, COLS), jnp.float32),
                jax.ShapeDtypeStruct((NP,), jnp.float32)]
    scratch = [
        pltpu.VMEM((N,), jnp.float32),
        pltpu.VMEM((N,), jnp.float32),
        pltpu.VMEM((SUBB, B), jnp.int32),
        pltpu.VMEM((SUBB, B), jnp.int32),
        pltpu.VMEM((SUBB, B), jnp.float32),
    ]
    if with_extras:
        out_type += [jax.ShapeDtypeStruct((NP,), jnp.float32)] * 3
        scratch += [pltpu.VMEM((SUBB, B), jnp.float32)]
    scratch += [
        pltpu.VMEM((B,), jnp.float32),
        pltpu.VMEM((B,), jnp.float32),
        pltpu.VMEM((B,), jnp.float32),
        pltpu.VMEM((B, COLS), jnp.float32),
        pltpu.VMEM((B, COLS), jnp.float32),
        pltpu.SemaphoreType.DMA,
        pltpu.SemaphoreType.DMA,
        pltpu.SemaphoreType.DMA,
        pltpu.SemaphoreType.DMA,
        pltpu.VMEM_SHARED((NP, COLS), jnp.float32),
        pltpu.VMEM_SHARED((NP,), jnp.float32),
    ]
    if with_extras:
        scratch += [pltpu.VMEM_SHARED((NP,), jnp.float32)] * 3
    return pl.kernel(
        functools.partial(_sc_body, with_extras),
        out_type=out_type,
        mesh=mesh,
        scratch_types=scratch,
        compiler_params=pltpu.CompilerParams(
            needs_layout_passes=False, use_tc_tiling_on_sc=False),
    )


def kernel(x, edge_index, edge_attr, W_node, b_node, W_eenc, b_eenc,
           W0, att_src0, att_dst0, We0, att_edge0, bias0, bn_w0, bn_b0,
           W1, att_src1, att_dst1, We1, att_edge1, bias1, bn_w1, bn_b1,
           W_out, b_out):
    src = edge_index[0]
    dst = edge_index[1]

    P0 = W_node @ W0
    bx0 = b_node @ W0
    us0 = P0 @ att_src0
    ud0 = P0 @ att_dst0
    z = jnp.zeros((128, 126), jnp.float32)
    wc0 = jnp.concatenate([P0, us0[:, None], ud0[:, None], z], axis=1)
    bc0 = jnp.concatenate(
        [bx0, jnp.stack([bx0 @ att_src0, bx0 @ att_dst0]),
         jnp.zeros((126,), jnp.float32)])

    we0 = We0 @ att_edge0
    we1 = We1 @ att_edge1
    gt0 = jnp.tile(W_eenc @ we0, 8).reshape(1, 128)
    gt1 = jnp.tile(W_eenc @ we1, 8).reshape(1, 128)
    smat = jnp.repeat(jnp.eye(8, dtype=jnp.float32), 16, axis=0)
    c0v = jnp.full((1, 8), b_eenc @ we0, jnp.float32)
    c1v = jnp.full((1, 8), b_eenc @ we1, jnp.float32)

    us1 = W1 @ att_src1
    ud1 = W1 @ att_dst1
    wc1 = jnp.concatenate([W1, us1[:, None], ud1[:, None], z], axis=1)
    bc1 = jnp.zeros((256,), jnp.float32)

    big0 = _mm(x, wc0, bc0, 2000)
    xs0 = big0[:, :128]
    asrc0 = big0[:, 128]
    adst0 = big0[:, 129]

    ea128 = edge_attr.reshape(E // 8, 128)
    a0, a1 = pl.pallas_call(
        _aer_body,
        grid=(25,),
        in_specs=(
            [pl.BlockSpec((1600, 128), lambda i: (i, 0))]
            + [pl.BlockSpec((1, 128), lambda i: (0, 0))] * 2
            + [pl.BlockSpec((128, 8), lambda i: (0, 0))]
            + [pl.BlockSpec((1, 8), lambda i: (0, 0))] * 2
        ),
        out_specs=[pl.BlockSpec((1600, 8), lambda i: (i, 0))] * 2,
        out_shape=[jax.ShapeDtypeStruct((E // 8, 8), jnp.float32)] * 2,
    )(ea128, gt0, gt1, smat, c0v, c1v)
    pad = ((0, EROWS - E // B), (0, 0))
    src_r = jnp.pad(src.reshape(E // B, B), pad)
    dst_r = jnp.pad(dst.reshape(E // B, B), pad)
    aer0_r = jnp.pad(a0.reshape(E // B, B), pad)
    aer1_r = jnp.pad(a1.reshape(E // B, B), pad)

    xs0f = jnp.concatenate([xs0[:, :COLS], xs0[:, COLS:]], axis=0)
    num0, den0, deg0, s0p, s1p = _sc_pass(True)(
        xs0f, asrc0, adst0, src_r, dst_r, aer0_r, aer1_r)

    r = lambda v: v.reshape(N, 1)
    rp = lambda v: v.reshape(NP, 1)
    big1 = pl.pallas_call(
        _merge_body,
        grid=(5,),
        in_specs=(
            [pl.BlockSpec((1, 2000, COLS), lambda i: (0, i, 0)),
             pl.BlockSpec((1, 2000, COLS), lambda i: (1, i, 0))]
            + [pl.BlockSpec((2000, 1), lambda i: (i, 0))] * 4
            + [pl.BlockSpec((2000, 1), lambda i: (i, 0))] * 2
            + [pl.BlockSpec((2000, 128), lambda i: (i, 0))]
            + [pl.BlockSpec((128, 256), lambda i: (0, 0)),
               pl.BlockSpec((1, 256), lambda i: (0, 0))]
            + [pl.BlockSpec((1, 128), lambda i: (0, 0))] * 3
        ),
        out_specs=pl.BlockSpec((2000, 256), lambda i: (i, 0)),
        out_shape=jax.ShapeDtypeStruct((N, 256), jnp.float32),
    )(num0, num0,
      rp(den0), rp(deg0), rp(s0p), rp(s1p),
      r(asrc0), r(adst0), xs0, wc1, bc1.reshape(1, 256),
      bias0.reshape(1, 128), bn_w0.reshape(1, 128), bn_b0.reshape(1, 128))

    xs1 = big1[:, :128]
    asrc1 = big1[:, 128]
    adst1 = big1[:, 129]
    pl1 = big1[:, 130]

    xs1f = jnp.concatenate([xs1[:, :COLS], xs1[:, COLS:]], axis=0)
    num1, den1 = _sc_pass(False)(xs1f, asrc1, adst1, src_r, dst_r, aer1_r)

    out = pl.pallas_call(
        _final_body,
        grid=(5,),
        in_specs=(
            [pl.BlockSpec((1, 2000, COLS), lambda i: (0, i, 0)),
             pl.BlockSpec((1, 2000, COLS), lambda i: (1, i, 0))]
            + [pl.BlockSpec((2000, 1), lambda i: (i, 0))] * 2
            + [pl.BlockSpec((2000, 128), lambda i: (i, 0))]
            + [pl.BlockSpec((1, 128), lambda i: (0, 0))] * 3
            + [pl.BlockSpec((128, 128), lambda i: (0, 0)),
               pl.BlockSpec((1, 128), lambda i: (0, 0))]
        ),
        out_specs=pl.BlockSpec((1, 128), lambda i: (0, 0)),
        out_shape=jax.ShapeDtypeStruct((1, 128), jnp.float32),
        scratch_shapes=[pltpu.VMEM((8, 128), jnp.float32)],
    )(num1, num1,
      rp(den1), r(pl1),
      xs1, bias1.reshape(1, 128), bn_w1.reshape(1, 128),
      bn_b1.reshape(1, 128), W_out, b_out.reshape(1, 128))

    return out.reshape(-1)
